# Initial kernel scaffold; baseline (speedup 1.0000x reference)
import jax
import jax.numpy as jnp
from jax.experimental import pallas as pl


def kernel(features, adjacency, W, b):
    raise NotImplementedError("write your pallas kernel here")


# speedup vs baseline: 1.8290x; 1.8290x over previous
import functools

import jax
import jax.numpy as jnp
from jax.experimental import pallas as pl
from jax.experimental.pallas import tpu as pltpu


def _assign_body(f_ref, w_ref, b_ref, out_ref):
    logits = jax.lax.dot_general(
        f_ref[...], w_ref[...],
        dimension_numbers=(((1,), (1,)), ((), ())),
        preferred_element_type=jnp.float32,
    ) + b_ref[...]
    m = jnp.max(logits, axis=1, keepdims=True)
    e = jnp.exp(logits - m)
    out_ref[...] = e / jnp.sum(e, axis=1, keepdims=True)


def _main_body(nblk, bm, a_ref, s_ref, loss_ref, d_acc, t_acc):
    i = pl.program_id(0)
    a = a_ref[...]
    s = s_ref[...]

    colsum = jnp.sum(a, axis=0, keepdims=True)
    m = jnp.dot(a, s, preferred_element_type=jnp.float32)
    s_blk = s_ref[pl.ds(i * bm, bm), :]
    part = jnp.sum(m * s_blk)

    @pl.when(i == 0)
    def _init():
        d_acc[...] = colsum
        t_acc[...] = jnp.full((1, 1), part, jnp.float32)

    @pl.when(i > 0)
    def _accum():
        d_acc[...] += colsum
        t_acc[...] += jnp.full((1, 1), part, jnp.float32)

    @pl.when(i == nblk - 1)
    def _finish():
        d = d_acc[...]
        edges = jnp.sum(d)
        std = jnp.dot(d, s, preferred_element_type=jnp.float32)
        trace_norm = jnp.sum(std * std) / (2.0 * edges)
        trace_gp = t_acc[0, 0]
        loss = -(trace_gp - trace_norm) / (2.0 * edges)
        loss_ref[...] = jnp.full((1, 1), loss, jnp.float32)


@functools.partial(jax.jit, static_argnames=())
def kernel(features, adjacency, W, b):
    n, d_feat = features.shape
    k = W.shape[0]

    assignments = pl.pallas_call(
        _assign_body,
        out_shape=jax.ShapeDtypeStruct((n, k), jnp.float32),
    )(features, W, b.reshape(1, k))

    bm = 400
    if n % bm != 0:
        bm = n
    nblk = n // bm

    loss = pl.pallas_call(
        functools.partial(_main_body, nblk, bm),
        grid=(nblk,),
        in_specs=[
            pl.BlockSpec((bm, n), lambda i: (i, 0)),
            pl.BlockSpec((n, k), lambda i: (0, 0)),
        ],
        out_specs=pl.BlockSpec((1, 1), lambda i: (0, 0)),
        out_shape=jax.ShapeDtypeStruct((1, 1), jnp.float32),
        scratch_shapes=[
            pltpu.VMEM((1, n), jnp.float32),
            pltpu.VMEM((1, 1), jnp.float32),
        ],
    )(adjacency, assignments)

    return assignments, loss[0, 0]


# speedup vs baseline: 1.8928x; 1.0349x over previous
import functools

import jax
import jax.numpy as jnp
from jax.experimental import pallas as pl
from jax.experimental.pallas import tpu as pltpu


def _body(nblk, bm, f_ref, w_ref, b_ref, a_ref, s_ref, loss_ref, d_acc, t_acc):
    i = pl.program_id(0)

    @pl.when(i == 0)
    def _assign():
        logits = jax.lax.dot_general(
            f_ref[...], w_ref[...],
            dimension_numbers=(((1,), (1,)), ((), ())),
            preferred_element_type=jnp.float32,
        ) + b_ref[...]
        mx = jnp.max(logits, axis=1, keepdims=True)
        e = jnp.exp(logits - mx)
        s_ref[...] = e / jnp.sum(e, axis=1, keepdims=True)

    a = a_ref[...]
    s = s_ref[...]

    colsum = jnp.sum(a, axis=0, keepdims=True)
    m = jnp.dot(a, s, preferred_element_type=jnp.float32)
    s_blk = s_ref[pl.ds(i * bm, bm), :]
    part = jnp.sum(m * s_blk)

    @pl.when(i == 0)
    def _init():
        d_acc[...] = colsum
        t_acc[...] = jnp.full((1, 1), part, jnp.float32)

    @pl.when(i > 0)
    def _accum():
        d_acc[...] += colsum
        t_acc[...] += jnp.full((1, 1), part, jnp.float32)

    @pl.when(i == nblk - 1)
    def _finish():
        d = d_acc[...]
        edges = jnp.sum(d)
        std = jnp.dot(d, s, preferred_element_type=jnp.float32)
        trace_norm = jnp.sum(std * std) / (2.0 * edges)
        loss = -(t_acc[0, 0] - trace_norm) / (2.0 * edges)
        loss_ref[...] = jnp.full((1, 1), loss, jnp.float32)


@jax.jit
def kernel(features, adjacency, W, b):
    n, d_feat = features.shape
    k = W.shape[0]

    bm = 400
    if n % bm != 0:
        bm = n
    nblk = n // bm

    assignments, loss = pl.pallas_call(
        functools.partial(_body, nblk, bm),
        grid=(nblk,),
        in_specs=[
            pl.BlockSpec((n, d_feat), lambda i: (0, 0)),
            pl.BlockSpec((k, d_feat), lambda i: (0, 0)),
            pl.BlockSpec((1, k), lambda i: (0, 0)),
            pl.BlockSpec((bm, n), lambda i: (i, 0)),
        ],
        out_specs=[
            pl.BlockSpec((n, k), lambda i: (0, 0)),
            pl.BlockSpec((1, 1), lambda i: (0, 0)),
        ],
        out_shape=[
            jax.ShapeDtypeStruct((n, k), jnp.float32),
            jax.ShapeDtypeStruct((1, 1), jnp.float32),
        ],
        scratch_shapes=[
            pltpu.VMEM((1, n), jnp.float32),
            pltpu.VMEM((1, 1), jnp.float32),
        ],
    )(features, W, b.reshape(1, k), adjacency)

    return assignments, loss[0, 0]
